# baseline (speedup 1.0000x reference)
import jax
import jax.numpy as jnp
from jax import lax
from jax.experimental import pallas as pl
from jax.experimental.pallas import tpu as pltpu

_NUM_CLASSES = 128
_H = 224
_W = 224
_P = _H * _W
_CB = 16
_NJ = _NUM_CLASSES // _CB
_K = 6


def _onehot_body(x_ref, out_ref, scratch, sem):
    nb = x_ref.shape[0]
    steps = nb * _NJ

    def _dst(s):
        b = s // _NJ
        j = lax.rem(s, _NJ)
        return out_ref.at[b, pl.ds(j * _CB, _CB), :]

    def _step(s, carry):
        slot = lax.rem(s, _K)

        @pl.when(s >= _K)
        def _():
            pltpu.make_async_copy(
                scratch.at[slot], _dst(s - _K), sem.at[slot]
            ).wait()

        b = s // _NJ
        j = lax.rem(s, _NJ)
        xv = x_ref[b]
        classes = j * _CB + lax.broadcasted_iota(jnp.int32, (_CB, 1), 0)
        scratch[slot] = (xv == classes).astype(jnp.float32)
        pltpu.make_async_copy(
            scratch.at[slot], _dst(s), sem.at[slot]
        ).start()
        return carry

    lax.fori_loop(0, steps, _step, 0)

    def _drain(k, carry):
        s = steps - _K + k
        slot = lax.rem(s, _K)
        pltpu.make_async_copy(
            scratch.at[slot], _dst(s), sem.at[slot]
        ).wait()
        return carry

    lax.fori_loop(0, _K, _drain, 0)


def kernel(x):
    b = x.shape[0]
    x3 = x.astype(jnp.int32).reshape(b, 1, _P)
    out = pl.pallas_call(
        _onehot_body,
        grid=(),
        in_specs=[pl.BlockSpec(memory_space=pltpu.VMEM)],
        out_specs=pl.BlockSpec(memory_space=pl.ANY),
        out_shape=jax.ShapeDtypeStruct((b, _NUM_CLASSES, _P), jnp.float32),
        scratch_shapes=[
            pltpu.VMEM((_K, _CB, _P), jnp.float32),
            pltpu.SemaphoreType.DMA((_K,)),
        ],
    )(x3)
    return out.reshape(b, _NUM_CLASSES, _H, _W)
